# baseline (speedup 1.0000x reference)
import jax
import jax.numpy as jnp
from jax.experimental import pallas as pl
from jax.experimental.pallas import tpu as pltpu
from jax.experimental.pallas import tpu_sc as plsc

S, D, H, E, K, F = 2048, 768, 12, 8, 2, 3072
DH = D // H
BT = 256
NT = S // BT
NEG = -1e30

GT = 256
CAP = S * K + E * GT
NTILES = CAP // GT
RT = 128
NRT = S // RT
SCW = 128


def _ln_qkv_body(x_ref, w_ref, ln_ref, qkv_ref):
    xf = x_ref[...]
    ms = jnp.mean(xf * xf, axis=-1, keepdims=True)
    xn = xf * jax.lax.rsqrt(ms + 1e-6) * ln_ref[...]
    qkv_ref[...] = jnp.dot(xn.astype(jnp.bfloat16), w_ref[...],
                           preferred_element_type=jnp.float32
                           ).astype(jnp.bfloat16)


def _ln_qkv(x2, wqkv_bf, ln1_w):
    return pl.pallas_call(
        _ln_qkv_body,
        grid=(NT,),
        in_specs=[
            pl.BlockSpec((BT, D), lambda i: (i, 0)),
            pl.BlockSpec((D, 3 * D), lambda i: (0, 0)),
            pl.BlockSpec((1, D), lambda i: (0, 0)),
        ],
        out_specs=pl.BlockSpec((BT, 3 * D), lambda i: (i, 0)),
        out_shape=jax.ShapeDtypeStruct((S, 3 * D), jnp.bfloat16),
    )(x2, wqkv_bf, ln1_w)


def _attn_body(qkv_ref, o_ref):
    i = pl.program_id(0)
    rows = i * BT + jax.lax.broadcasted_iota(jnp.int32, (BT, S), 0)
    cols = jax.lax.broadcasted_iota(jnp.int32, (BT, S), 1)
    causal = cols <= rows
    outs = []
    for h in range(H):
        q = qkv_ref[pl.ds(i * BT, BT), h * DH:(h + 1) * DH]
        k = qkv_ref[:, D + h * DH:D + (h + 1) * DH]
        v = qkv_ref[:, 2 * D + h * DH:2 * D + (h + 1) * DH]
        s = jax.lax.dot_general(q.astype(jnp.bfloat16),
                                k.astype(jnp.bfloat16),
                                (((1,), (1,)), ((), ())),
                                preferred_element_type=jnp.float32)
        s = s * (1.0 / 8.0)
        s = jnp.where(causal, s, NEG)
        m = jnp.max(s, axis=-1, keepdims=True)
        p = jnp.exp(s - m)
        p = p / jnp.sum(p, axis=-1, keepdims=True)
        outs.append(jnp.dot(p.astype(jnp.bfloat16), v.astype(jnp.bfloat16),
                            preferred_element_type=jnp.float32))
    o_ref[...] = jnp.concatenate(outs, axis=1)


def _attn(qkv):
    return pl.pallas_call(
        _attn_body,
        grid=(NT,),
        in_specs=[pl.BlockSpec((S, 3 * D), lambda i: (0, 0))],
        out_specs=pl.BlockSpec((BT, D), lambda i: (i, 0)),
        out_shape=jax.ShapeDtypeStruct((S, D), jnp.float32),
    )(qkv)


def _proj_route_body(o_ref, x_ref, wo_ref, ln_ref, gw_ref,
                     a_ref, h_ref, route_ref):
    a = x_ref[...] + jnp.dot(o_ref[...].astype(jnp.bfloat16), wo_ref[...],
                             preferred_element_type=jnp.float32)
    a_ref[...] = a
    ms = jnp.mean(a * a, axis=-1, keepdims=True)
    hn = a * jax.lax.rsqrt(ms + 1e-6) * ln_ref[...]
    h_ref[...] = hn
    logits = jnp.dot(hn.astype(jnp.bfloat16), gw_ref[...],
                     preferred_element_type=jnp.float32)
    lane = jax.lax.broadcasted_iota(jnp.int32, (BT, 128), 1)
    logits = jnp.where(lane < E, logits, NEG)
    m1 = jnp.max(logits, axis=-1, keepdims=True)
    idx1 = jnp.min(jnp.where(logits == m1, lane, 127), axis=-1, keepdims=True)
    oh1 = (lane == idx1).astype(jnp.float32)
    lm = jnp.where(lane == idx1, NEG, logits)
    m2 = jnp.max(lm, axis=-1, keepdims=True)
    idx2 = jnp.min(jnp.where(lm == m2, lane, 127), axis=-1, keepdims=True)
    oh2 = (lane == idx2).astype(jnp.float32)
    d = jnp.exp(m2 - m1)
    p1 = 1.0 / (1.0 + d)
    p2 = d / (1.0 + d)
    route_ref[...] = (jnp.where(lane == 0, idx1.astype(jnp.float32), 0.0)
                      + jnp.where(lane == 1, idx2.astype(jnp.float32), 0.0)
                      + jnp.where(lane == 2, p1, 0.0)
                      + jnp.where(lane == 3, p2, 0.0))


def _proj_route(o, x2, wo_bf, ln2_w, gate_pad):
    return pl.pallas_call(
        _proj_route_body,
        grid=(NT,),
        in_specs=[
            pl.BlockSpec((BT, D), lambda i: (i, 0)),
            pl.BlockSpec((BT, D), lambda i: (i, 0)),
            pl.BlockSpec((D, D), lambda i: (0, 0)),
            pl.BlockSpec((1, D), lambda i: (0, 0)),
            pl.BlockSpec((D, 128), lambda i: (0, 0)),
        ],
        out_specs=[
            pl.BlockSpec((BT, D), lambda i: (i, 0)),
            pl.BlockSpec((BT, D), lambda i: (i, 0)),
            pl.BlockSpec((BT, 128), lambda i: (i, 0)),
        ],
        out_shape=[
            jax.ShapeDtypeStruct((S, D), jnp.float32),
            jax.ShapeDtypeStruct((S, D), jnp.float32),
            jax.ShapeDtypeStruct((S, 128), jnp.float32),
        ],
    )(o, x2, wo_bf, ln2_w, gate_pad)


def _ranks_body(route_ref, ranks_ref, meta_ref, carry_ref):
    i = pl.program_id(0)

    @pl.when(i == 0)
    def _():
        carry_ref[...] = jnp.zeros((8, 128), jnp.float32)

    lane = jax.lax.broadcasted_iota(jnp.int32, (RT, 128), 1)
    lanef = lane.astype(jnp.float32)
    r = route_ref[...]
    idx0 = r[:, 0:1]
    idx1 = r[:, 1:2]
    oh0 = (lanef == idx0).astype(jnp.float32)
    oh1 = (lanef == idx1).astype(jnp.float32)
    ohb = oh0 + oh1
    row_i = jax.lax.broadcasted_iota(jnp.int32, (RT, RT), 0)
    col_i = jax.lax.broadcasted_iota(jnp.int32, (RT, RT), 1)
    ltri = (col_i <= row_i).astype(jnp.float32)
    incl = jnp.dot(ltri, ohb, preferred_element_type=jnp.float32)
    carry = carry_ref[0:1, :]
    cb = carry + incl - ohb
    rank0 = jnp.sum(cb * oh0, axis=-1, keepdims=True)
    rank1 = jnp.sum(cb * oh1, axis=-1, keepdims=True)
    ranks_ref[...] = (jnp.where(lane == 0, rank0, 0.0)
                      + jnp.where(lane == 1, rank1, 0.0))
    counts = carry + jnp.sum(ohb, axis=0, keepdims=True)
    carry_ref[0:1, :] = counts

    @pl.when(i == NRT - 1)
    def _():
        padded = jnp.floor((counts + (GT - 1)) * (1.0 / GT)) * GT
        ru = jax.lax.broadcasted_iota(jnp.int32, (128, 128), 0)
        cu = jax.lax.broadcasted_iota(jnp.int32, (128, 128), 1)
        sut = (ru < cu).astype(jnp.float32)
        off = jnp.dot(padded, sut, preferred_element_type=jnp.float32)
        ends = off + padded
        sub = jax.lax.broadcasted_iota(jnp.int32, (8, 128), 0)
        meta_ref[...] = (jnp.where(sub == 0, off, 0.0)
                         + jnp.where(sub == 1, ends, 0.0))


def _ranks(route):
    return pl.pallas_call(
        _ranks_body,
        grid=(NRT,),
        in_specs=[pl.BlockSpec((RT, 128), lambda i: (i, 0))],
        out_specs=[
            pl.BlockSpec((RT, 128), lambda i: (i, 0)),
            pl.BlockSpec((8, 128), lambda i: (0, 0)),
        ],
        out_shape=[
            jax.ShapeDtypeStruct((S, 128), jnp.float32),
            jax.ShapeDtypeStruct((8, 128), jnp.float32),
        ],
        scratch_shapes=[pltpu.VMEM((8, 128), jnp.float32)],
    )(route)


def _dest_body(route_ref, ranks_ref, meta_ref, dest_ref):
    lane = jax.lax.broadcasted_iota(jnp.int32, (S, 128), 1)
    lanef = lane.astype(jnp.float32)
    r = route_ref[...]
    idx0 = r[:, 0:1]
    idx1 = r[:, 1:2]
    off_row = meta_ref[0:1, :]
    off0 = jnp.sum((lanef == idx0).astype(jnp.float32) * off_row,
                   axis=-1, keepdims=True)
    off1 = jnp.sum((lanef == idx1).astype(jnp.float32) * off_row,
                   axis=-1, keepdims=True)
    dest0 = off0 + ranks_ref[:, 0:1]
    dest1 = off1 + ranks_ref[:, 1:2]
    dest_ref[...] = (jnp.where(lane == 0, dest0, 0.0)
                     + jnp.where(lane == 1, dest1, 0.0))


def _dest(route, ranks, meta):
    return pl.pallas_call(
        _dest_body,
        grid=(1,),
        in_specs=[
            pl.BlockSpec((S, 128), lambda i: (0, 0)),
            pl.BlockSpec((S, 128), lambda i: (0, 0)),
            pl.BlockSpec((8, 128), lambda i: (0, 0)),
        ],
        out_specs=pl.BlockSpec((S, 128), lambda i: (0, 0)),
        out_shape=jax.ShapeDtypeStruct((S, 128), jnp.float32),
    )(route, ranks, meta)


def _vector_mesh():
    return plsc.VectorSubcoreMesh(core_axis_name="core",
                                  subcore_axis_name="subcore")


HD = D // 2


def _sc_scatter_h(h2, idx2):
    @pl.kernel(out_type=jax.ShapeDtypeStruct((2 * CAP, HD), jnp.float32),
               mesh=_vector_mesh())
    def k(h_hbm, i_hbm, o_hbm):
        def body(h_vmem, i_vmem):
            pltpu.sync_copy(h_vmem, o_hbm.at[i_vmem.at[0]])

        pltpu.emit_pipeline(
            body,
            grid=(K, 2 * S // SCW),
            in_specs=[
                pl.BlockSpec((SCW, HD), index_map=lambda s, c: (c, 0)),
                pl.BlockSpec((1, SCW), index_map=lambda s, c: (s, c)),
            ],
            out_specs=[],
            core_axis_name=("core", "subcore"),
            dimension_semantics=(pltpu.PARALLEL, pltpu.PARALLEL),
        )(h_hbm, i_hbm)

    return k(h2, idx2)


FC = 768


def _group_ffn_body(et_ref, hs_ref, w1_ref, w2_ref, y_ref):
    hb = hs_ref[...].astype(jnp.bfloat16)
    acc = None
    for c in range(F // FC):
        t = jnp.dot(hb, w1_ref[0, :, c * FC:(c + 1) * FC],
                    preferred_element_type=jnp.float32)
        act = t * jax.nn.sigmoid(t)
        part = jnp.dot(act.astype(jnp.bfloat16),
                       w2_ref[0, c * FC:(c + 1) * FC, :],
                       preferred_element_type=jnp.float32)
        acc = part if acc is None else acc + part
    y_ref[...] = acc


def _group_ffn(etile, hs, w1_bf, w2_bf):
    grid_spec = pltpu.PrefetchScalarGridSpec(
        num_scalar_prefetch=1,
        grid=(NTILES,),
        in_specs=[
            pl.BlockSpec((GT, D), lambda i, et: (i, 0)),
            pl.BlockSpec((1, D, F), lambda i, et: (et[i], 0, 0)),
            pl.BlockSpec((1, F, D), lambda i, et: (et[i], 0, 0)),
        ],
        out_specs=pl.BlockSpec((GT, D), lambda i, et: (i, 0)),
    )
    return pl.pallas_call(
        _group_ffn_body,
        grid_spec=grid_spec,
        out_shape=jax.ShapeDtypeStruct((CAP, D), jnp.float32),
    )(etile, hs, w1_bf, w2_bf)


def _sc_gather_y(y2, didx):
    n = 2 * S * K

    @pl.kernel(out_type=jax.ShapeDtypeStruct((n, HD), jnp.float32),
               mesh=_vector_mesh())
    def k(y_hbm, i_hbm, o_hbm):
        def body(i_vmem, o_vmem):
            pltpu.sync_copy(y_hbm.at[i_vmem.at[0]], o_vmem)

        pltpu.emit_pipeline(
            body,
            grid=(n // SCW,),
            in_specs=[pl.BlockSpec((1, SCW), index_map=lambda i: (0, i))],
            out_specs=[pl.BlockSpec((SCW, HD), index_map=lambda i: (i, 0))],
            core_axis_name=("core", "subcore"),
            dimension_semantics=(pltpu.PARALLEL,),
        )(i_hbm, o_hbm)

    return k(y2, didx)


def _combine_body(a_ref, yc_ref, route_ref, out_ref):
    lane = jax.lax.broadcasted_iota(jnp.int32, (BT, 128), 1)
    r = route_ref[...]
    p0 = jnp.sum(jnp.where(lane == 2, r, 0.0), axis=-1, keepdims=True)
    p1 = jnp.sum(jnp.where(lane == 3, r, 0.0), axis=-1, keepdims=True)
    yc = yc_ref[...]
    out_ref[...] = a_ref[...] + p0 * yc[:, :D] + p1 * yc[:, D:]


def _combine(a, yc2, route):
    return pl.pallas_call(
        _combine_body,
        grid=(NT,),
        in_specs=[
            pl.BlockSpec((BT, D), lambda i: (i, 0)),
            pl.BlockSpec((BT, K * D), lambda i: (i, 0)),
            pl.BlockSpec((BT, 128), lambda i: (i, 0)),
        ],
        out_specs=pl.BlockSpec((BT, D), lambda i: (i, 0)),
        out_shape=jax.ShapeDtypeStruct((S, D), jnp.float32),
    )(a, yc2, route)


def kernel(x, ln1_w, ln2_w, Wq, Wk, Wv, Wo, gate_w, W1, W2):
    x2 = x.reshape(S, D)
    wqkv = jnp.concatenate([Wq, Wk, Wv], axis=1).astype(jnp.bfloat16)
    qkv = _ln_qkv(x2, wqkv, ln1_w.reshape(1, D))
    o = _attn(qkv)
    gate_pad = jnp.pad(gate_w, ((0, 0), (0, 128 - E))).astype(jnp.bfloat16)
    a, h_bf, route = _proj_route(o, x2, Wo.astype(jnp.bfloat16),
                                 ln2_w.reshape(1, D), gate_pad)
    ranks, meta = _ranks(route)
    dest = _dest(route, ranks, meta)
    dest_i = dest[:, :K].astype(jnp.int32)
    ends = meta[1, :E]
    starts = jnp.arange(NTILES, dtype=jnp.float32) * GT
    etile = jnp.minimum(
        jnp.sum((starts[:, None] >= ends[None, :]).astype(jnp.int32), axis=1),
        E - 1).astype(jnp.int32)
    slot = dest_i.T
    idx2 = jnp.stack([2 * slot, 2 * slot + 1], axis=-1).reshape(K, 2 * S)
    hs2 = _sc_scatter_h(h_bf.reshape(2 * S, HD), idx2)
    ys = _group_ffn(etile, hs2.reshape(CAP, D), W1.astype(jnp.bfloat16),
                    W2.astype(jnp.bfloat16))
    dflat = dest_i.reshape(S * K)
    didx = jnp.stack([2 * dflat, 2 * dflat + 1], axis=-1).reshape(1, 2 * S * K)
    yc = _sc_gather_y(ys.reshape(2 * CAP, HD), didx)
    out = _combine(a, yc.reshape(S, K * D), route)
    return out.reshape(1, S, D)


# speedup vs baseline: 1.0318x; 1.0318x over previous
import jax
import jax.numpy as jnp
from jax.experimental import pallas as pl
from jax.experimental.pallas import tpu as pltpu
from jax.experimental.pallas import tpu_sc as plsc

S, D, H, E, K, F = 2048, 768, 12, 8, 2, 3072
DH = D // H
BT = 256
NT = S // BT
NEG = -1e30

GT = 256
CAP = S * K + E * GT
NTILES = CAP // GT
RT = 128
NRT = S // RT
SCW = 128


def _ln_qkv_body(x_ref, w_ref, ln_ref, qkv_ref):
    xf = x_ref[...]
    ms = jnp.mean(xf * xf, axis=-1, keepdims=True)
    xn = xf * jax.lax.rsqrt(ms + 1e-6) * ln_ref[...]
    qkv_ref[...] = jnp.dot(xn.astype(jnp.bfloat16), w_ref[...],
                           preferred_element_type=jnp.float32
                           ).astype(jnp.bfloat16)


def _ln_qkv(x2, wqkv_bf, ln1_w):
    return pl.pallas_call(
        _ln_qkv_body,
        grid=(NT,),
        in_specs=[
            pl.BlockSpec((BT, D), lambda i: (i, 0)),
            pl.BlockSpec((D, 3 * D), lambda i: (0, 0)),
            pl.BlockSpec((1, D), lambda i: (0, 0)),
        ],
        out_specs=pl.BlockSpec((BT, 3 * D), lambda i: (i, 0)),
        out_shape=jax.ShapeDtypeStruct((S, 3 * D), jnp.bfloat16),
    )(x2, wqkv_bf, ln1_w)


def _attn_body(qkv_ref, o_ref, *, base, sk):
    i = pl.program_id(0)
    rows = base + i * BT + jax.lax.broadcasted_iota(jnp.int32, (BT, sk), 0)
    cols = jax.lax.broadcasted_iota(jnp.int32, (BT, sk), 1)
    causal = cols <= rows
    outs = []
    for h in range(H):
        q = qkv_ref[pl.ds(base + i * BT, BT), h * DH:(h + 1) * DH]
        k = qkv_ref[pl.ds(0, sk), D + h * DH:D + (h + 1) * DH]
        v = qkv_ref[pl.ds(0, sk), 2 * D + h * DH:2 * D + (h + 1) * DH]
        s = jax.lax.dot_general(q.astype(jnp.bfloat16),
                                k.astype(jnp.bfloat16),
                                (((1,), (1,)), ((), ())),
                                preferred_element_type=jnp.float32)
        s = s * (1.0 / 8.0)
        s = jnp.where(causal, s, NEG)
        m = jnp.max(s, axis=-1, keepdims=True)
        p = jnp.exp(s - m)
        p = p / jnp.sum(p, axis=-1, keepdims=True)
        outs.append(jnp.dot(p.astype(jnp.bfloat16), v.astype(jnp.bfloat16),
                            preferred_element_type=jnp.float32))
    o_ref[...] = jnp.concatenate(outs, axis=1)


def _attn_half(qkv, base, sk):
    import functools
    nrow = S // 2
    return pl.pallas_call(
        functools.partial(_attn_body, base=base, sk=sk),
        grid=(nrow // BT,),
        in_specs=[pl.BlockSpec((S, 3 * D), lambda i: (0, 0))],
        out_specs=pl.BlockSpec((BT, D), lambda i: (i, 0)),
        out_shape=jax.ShapeDtypeStruct((nrow, D), jnp.float32),
    )(qkv)


def _attn(qkv):
    lo = _attn_half(qkv, 0, S // 2)
    hi = _attn_half(qkv, S // 2, S)
    return jnp.concatenate([lo, hi], axis=0)


def _proj_route_body(o_ref, x_ref, wo_ref, ln_ref, gw_ref,
                     a_ref, h_ref, route_ref):
    a = x_ref[...] + jnp.dot(o_ref[...].astype(jnp.bfloat16), wo_ref[...],
                             preferred_element_type=jnp.float32)
    a_ref[...] = a
    ms = jnp.mean(a * a, axis=-1, keepdims=True)
    hn = a * jax.lax.rsqrt(ms + 1e-6) * ln_ref[...]
    h_ref[...] = hn
    logits = jnp.dot(hn.astype(jnp.bfloat16), gw_ref[...],
                     preferred_element_type=jnp.float32)
    lane = jax.lax.broadcasted_iota(jnp.int32, (BT, 128), 1)
    logits = jnp.where(lane < E, logits, NEG)
    m1 = jnp.max(logits, axis=-1, keepdims=True)
    idx1 = jnp.min(jnp.where(logits == m1, lane, 127), axis=-1, keepdims=True)
    oh1 = (lane == idx1).astype(jnp.float32)
    lm = jnp.where(lane == idx1, NEG, logits)
    m2 = jnp.max(lm, axis=-1, keepdims=True)
    idx2 = jnp.min(jnp.where(lm == m2, lane, 127), axis=-1, keepdims=True)
    oh2 = (lane == idx2).astype(jnp.float32)
    d = jnp.exp(m2 - m1)
    p1 = 1.0 / (1.0 + d)
    p2 = d / (1.0 + d)
    route_ref[...] = (jnp.where(lane == 0, idx1.astype(jnp.float32), 0.0)
                      + jnp.where(lane == 1, idx2.astype(jnp.float32), 0.0)
                      + jnp.where(lane == 2, p1, 0.0)
                      + jnp.where(lane == 3, p2, 0.0))


def _proj_route(o, x2, wo_bf, ln2_w, gate_pad):
    return pl.pallas_call(
        _proj_route_body,
        grid=(NT,),
        in_specs=[
            pl.BlockSpec((BT, D), lambda i: (i, 0)),
            pl.BlockSpec((BT, D), lambda i: (i, 0)),
            pl.BlockSpec((D, D), lambda i: (0, 0)),
            pl.BlockSpec((1, D), lambda i: (0, 0)),
            pl.BlockSpec((D, 128), lambda i: (0, 0)),
        ],
        out_specs=[
            pl.BlockSpec((BT, D), lambda i: (i, 0)),
            pl.BlockSpec((BT, D), lambda i: (i, 0)),
            pl.BlockSpec((BT, 128), lambda i: (i, 0)),
        ],
        out_shape=[
            jax.ShapeDtypeStruct((S, D), jnp.float32),
            jax.ShapeDtypeStruct((S, D), jnp.float32),
            jax.ShapeDtypeStruct((S, 128), jnp.float32),
        ],
    )(o, x2, wo_bf, ln2_w, gate_pad)


def _ranks_body(route_ref, ranks_ref, meta_ref, carry_ref):
    i = pl.program_id(0)

    @pl.when(i == 0)
    def _():
        carry_ref[...] = jnp.zeros((8, 128), jnp.float32)

    lane = jax.lax.broadcasted_iota(jnp.int32, (RT, 128), 1)
    lanef = lane.astype(jnp.float32)
    r = route_ref[...]
    idx0 = r[:, 0:1]
    idx1 = r[:, 1:2]
    oh0 = (lanef == idx0).astype(jnp.float32)
    oh1 = (lanef == idx1).astype(jnp.float32)
    ohb = oh0 + oh1
    row_i = jax.lax.broadcasted_iota(jnp.int32, (RT, RT), 0)
    col_i = jax.lax.broadcasted_iota(jnp.int32, (RT, RT), 1)
    ltri = (col_i <= row_i).astype(jnp.float32)
    incl = jnp.dot(ltri, ohb, preferred_element_type=jnp.float32)
    carry = carry_ref[0:1, :]
    cb = carry + incl - ohb
    rank0 = jnp.sum(cb * oh0, axis=-1, keepdims=True)
    rank1 = jnp.sum(cb * oh1, axis=-1, keepdims=True)
    ranks_ref[...] = (jnp.where(lane == 0, rank0, 0.0)
                      + jnp.where(lane == 1, rank1, 0.0))
    counts = carry + jnp.sum(ohb, axis=0, keepdims=True)
    carry_ref[0:1, :] = counts

    @pl.when(i == NRT - 1)
    def _():
        padded = jnp.floor((counts + (GT - 1)) * (1.0 / GT)) * GT
        ru = jax.lax.broadcasted_iota(jnp.int32, (128, 128), 0)
        cu = jax.lax.broadcasted_iota(jnp.int32, (128, 128), 1)
        sut = (ru < cu).astype(jnp.float32)
        off = jnp.dot(padded, sut, preferred_element_type=jnp.float32)
        ends = off + padded
        sub = jax.lax.broadcasted_iota(jnp.int32, (8, 128), 0)
        meta_ref[...] = (jnp.where(sub == 0, off, 0.0)
                         + jnp.where(sub == 1, ends, 0.0))


def _ranks(route):
    return pl.pallas_call(
        _ranks_body,
        grid=(NRT,),
        in_specs=[pl.BlockSpec((RT, 128), lambda i: (i, 0))],
        out_specs=[
            pl.BlockSpec((RT, 128), lambda i: (i, 0)),
            pl.BlockSpec((8, 128), lambda i: (0, 0)),
        ],
        out_shape=[
            jax.ShapeDtypeStruct((S, 128), jnp.float32),
            jax.ShapeDtypeStruct((8, 128), jnp.float32),
        ],
        scratch_shapes=[pltpu.VMEM((8, 128), jnp.float32)],
    )(route)


def _dest_body(route_ref, ranks_ref, meta_ref, dest_ref):
    lane = jax.lax.broadcasted_iota(jnp.int32, (S, 128), 1)
    lanef = lane.astype(jnp.float32)
    r = route_ref[...]
    idx0 = r[:, 0:1]
    idx1 = r[:, 1:2]
    off_row = meta_ref[0:1, :]
    off0 = jnp.sum((lanef == idx0).astype(jnp.float32) * off_row,
                   axis=-1, keepdims=True)
    off1 = jnp.sum((lanef == idx1).astype(jnp.float32) * off_row,
                   axis=-1, keepdims=True)
    dest0 = off0 + ranks_ref[:, 0:1]
    dest1 = off1 + ranks_ref[:, 1:2]
    dest_ref[...] = (jnp.where(lane == 0, dest0, 0.0)
                     + jnp.where(lane == 1, dest1, 0.0))


def _dest(route, ranks, meta):
    return pl.pallas_call(
        _dest_body,
        grid=(1,),
        in_specs=[
            pl.BlockSpec((S, 128), lambda i: (0, 0)),
            pl.BlockSpec((S, 128), lambda i: (0, 0)),
            pl.BlockSpec((8, 128), lambda i: (0, 0)),
        ],
        out_specs=pl.BlockSpec((S, 128), lambda i: (0, 0)),
        out_shape=jax.ShapeDtypeStruct((S, 128), jnp.float32),
    )(route, ranks, meta)


def _vector_mesh():
    return plsc.VectorSubcoreMesh(core_axis_name="core",
                                  subcore_axis_name="subcore")


HD = D // 2


def _sc_scatter_h(h2, idx2):
    @pl.kernel(out_type=jax.ShapeDtypeStruct((2 * CAP, HD), jnp.float32),
               mesh=_vector_mesh())
    def k(h_hbm, i_hbm, o_hbm):
        def body(h_vmem, i_vmem):
            pltpu.sync_copy(h_vmem, o_hbm.at[i_vmem.at[0]])

        pltpu.emit_pipeline(
            body,
            grid=(K, 2 * S // SCW),
            in_specs=[
                pl.BlockSpec((SCW, HD), index_map=lambda s, c: (c, 0)),
                pl.BlockSpec((1, SCW), index_map=lambda s, c: (s, c)),
            ],
            out_specs=[],
            core_axis_name=("core", "subcore"),
            dimension_semantics=(pltpu.PARALLEL, pltpu.PARALLEL),
        )(h_hbm, i_hbm)

    return k(h2, idx2)


FC = 768


def _group_ffn_body(et_ref, hs_ref, w1_ref, w2_ref, y_ref):
    hb = hs_ref[...].astype(jnp.bfloat16)
    acc = None
    for c in range(F // FC):
        t = jnp.dot(hb, w1_ref[0, :, c * FC:(c + 1) * FC],
                    preferred_element_type=jnp.float32)
        act = t * jax.nn.sigmoid(t)
        part = jnp.dot(act.astype(jnp.bfloat16),
                       w2_ref[0, c * FC:(c + 1) * FC, :],
                       preferred_element_type=jnp.float32)
        acc = part if acc is None else acc + part
    y_ref[...] = acc


def _group_ffn(etile, hs, w1_bf, w2_bf):
    grid_spec = pltpu.PrefetchScalarGridSpec(
        num_scalar_prefetch=1,
        grid=(NTILES,),
        in_specs=[
            pl.BlockSpec((GT, D), lambda i, et: (i, 0)),
            pl.BlockSpec((1, D, F), lambda i, et: (et[i], 0, 0)),
            pl.BlockSpec((1, F, D), lambda i, et: (et[i], 0, 0)),
        ],
        out_specs=pl.BlockSpec((GT, D), lambda i, et: (i, 0)),
    )
    return pl.pallas_call(
        _group_ffn_body,
        grid_spec=grid_spec,
        out_shape=jax.ShapeDtypeStruct((CAP, D), jnp.float32),
    )(etile, hs, w1_bf, w2_bf)


def _sc_gather_y(y2, didx):
    n = 2 * S * K

    @pl.kernel(out_type=jax.ShapeDtypeStruct((n, HD), jnp.float32),
               mesh=_vector_mesh())
    def k(y_hbm, i_hbm, o_hbm):
        def body(i_vmem, o_vmem):
            pltpu.sync_copy(y_hbm.at[i_vmem.at[0]], o_vmem)

        pltpu.emit_pipeline(
            body,
            grid=(n // SCW,),
            in_specs=[pl.BlockSpec((1, SCW), index_map=lambda i: (0, i))],
            out_specs=[pl.BlockSpec((SCW, HD), index_map=lambda i: (i, 0))],
            core_axis_name=("core", "subcore"),
            dimension_semantics=(pltpu.PARALLEL,),
        )(i_hbm, o_hbm)

    return k(y2, didx)


def _combine_body(a_ref, yc_ref, route_ref, out_ref):
    lane = jax.lax.broadcasted_iota(jnp.int32, (BT, 128), 1)
    r = route_ref[...]
    p0 = jnp.sum(jnp.where(lane == 2, r, 0.0), axis=-1, keepdims=True)
    p1 = jnp.sum(jnp.where(lane == 3, r, 0.0), axis=-1, keepdims=True)
    yc = yc_ref[...]
    out_ref[...] = a_ref[...] + p0 * yc[:, :D] + p1 * yc[:, D:]


def _combine(a, yc2, route):
    return pl.pallas_call(
        _combine_body,
        grid=(NT,),
        in_specs=[
            pl.BlockSpec((BT, D), lambda i: (i, 0)),
            pl.BlockSpec((BT, K * D), lambda i: (i, 0)),
            pl.BlockSpec((BT, 128), lambda i: (i, 0)),
        ],
        out_specs=pl.BlockSpec((BT, D), lambda i: (i, 0)),
        out_shape=jax.ShapeDtypeStruct((S, D), jnp.float32),
    )(a, yc2, route)


def kernel(x, ln1_w, ln2_w, Wq, Wk, Wv, Wo, gate_w, W1, W2):
    x2 = x.reshape(S, D)
    wqkv = jnp.concatenate([Wq, Wk, Wv], axis=1).astype(jnp.bfloat16)
    qkv = _ln_qkv(x2, wqkv, ln1_w.reshape(1, D))
    o = _attn(qkv)
    gate_pad = jnp.pad(gate_w, ((0, 0), (0, 128 - E))).astype(jnp.bfloat16)
    a, h_bf, route = _proj_route(o, x2, Wo.astype(jnp.bfloat16),
                                 ln2_w.reshape(1, D), gate_pad)
    ranks, meta = _ranks(route)
    dest = _dest(route, ranks, meta)
    dest_i = dest[:, :K].astype(jnp.int32)
    ends = meta[1, :E]
    starts = jnp.arange(NTILES, dtype=jnp.float32) * GT
    etile = jnp.minimum(
        jnp.sum((starts[:, None] >= ends[None, :]).astype(jnp.int32), axis=1),
        E - 1).astype(jnp.int32)
    slot = dest_i.T
    idx2 = jnp.stack([2 * slot, 2 * slot + 1], axis=-1).reshape(K, 2 * S)
    hs2 = _sc_scatter_h(h_bf.reshape(2 * S, HD), idx2)
    ys = _group_ffn(etile, hs2.reshape(CAP, D), W1.astype(jnp.bfloat16),
                    W2.astype(jnp.bfloat16))
    dflat = dest_i.reshape(S * K)
    didx = jnp.stack([2 * dflat, 2 * dflat + 1], axis=-1).reshape(1, 2 * S * K)
    yc = _sc_gather_y(ys.reshape(2 * CAP, HD), didx)
    out = _combine(a, yc.reshape(S, K * D), route)
    return out.reshape(1, S, D)


# speedup vs baseline: 1.1577x; 1.1220x over previous
import jax
import jax.numpy as jnp
from jax.experimental import pallas as pl
from jax.experimental.pallas import tpu as pltpu
from jax.experimental.pallas import tpu_sc as plsc

S, D, H, E, K, F = 2048, 768, 12, 8, 2, 3072
DH = D // H
BT = 256
NT = S // BT
NEG = -1e30

GT = 256
CAP = S * K + E * GT
NTILES = CAP // GT
RT = 128
NRT = S // RT
SCW = 128


def _ln_qkv_body(x_ref, w_ref, ln_ref, qkv_ref):
    xf = x_ref[...]
    ms = jnp.mean(xf * xf, axis=-1, keepdims=True)
    xn = xf * jax.lax.rsqrt(ms + 1e-6) * ln_ref[...]
    qkv_ref[...] = jnp.dot(xn.astype(jnp.bfloat16), w_ref[...],
                           preferred_element_type=jnp.float32
                           ).astype(jnp.bfloat16)


def _ln_qkv(x2, wqkv_bf, ln1_w):
    return pl.pallas_call(
        _ln_qkv_body,
        grid=(NT,),
        in_specs=[
            pl.BlockSpec((BT, D), lambda i: (i, 0)),
            pl.BlockSpec((D, 3 * D), lambda i: (0, 0)),
            pl.BlockSpec((1, D), lambda i: (0, 0)),
        ],
        out_specs=pl.BlockSpec((BT, 3 * D), lambda i: (i, 0)),
        out_shape=jax.ShapeDtypeStruct((S, 3 * D), jnp.bfloat16),
    )(x2, wqkv_bf, ln1_w)


def _attn_body(qkv_ref, o_ref, *, base, sk):
    i = pl.program_id(0)
    rows = base + i * BT + jax.lax.broadcasted_iota(jnp.int32, (BT, sk), 0)
    cols = jax.lax.broadcasted_iota(jnp.int32, (BT, sk), 1)
    causal = cols <= rows
    outs = []
    for h in range(H):
        q = qkv_ref[pl.ds(base + i * BT, BT), h * DH:(h + 1) * DH]
        k = qkv_ref[pl.ds(0, sk), D + h * DH:D + (h + 1) * DH]
        v = qkv_ref[pl.ds(0, sk), 2 * D + h * DH:2 * D + (h + 1) * DH]
        s = jax.lax.dot_general(q.astype(jnp.bfloat16),
                                k.astype(jnp.bfloat16),
                                (((1,), (1,)), ((), ())),
                                preferred_element_type=jnp.float32)
        s = s * (1.0 / 8.0)
        s = jnp.where(causal, s, NEG)
        m = jnp.max(s, axis=-1, keepdims=True)
        p = jnp.exp(s - m)
        p = p / jnp.sum(p, axis=-1, keepdims=True)
        outs.append(jnp.dot(p.astype(jnp.bfloat16), v.astype(jnp.bfloat16),
                            preferred_element_type=jnp.float32))
    o_ref[...] = jnp.concatenate(outs, axis=1)


def _attn_half(qkv, base, sk):
    import functools
    nrow = S // 2
    return pl.pallas_call(
        functools.partial(_attn_body, base=base, sk=sk),
        grid=(nrow // BT,),
        in_specs=[pl.BlockSpec((S, 3 * D), lambda i: (0, 0))],
        out_specs=pl.BlockSpec((BT, D), lambda i: (i, 0)),
        out_shape=jax.ShapeDtypeStruct((nrow, D), jnp.float32),
    )(qkv)


def _attn(qkv):
    lo = _attn_half(qkv, 0, S // 2)
    hi = _attn_half(qkv, S // 2, S)
    return jnp.concatenate([lo, hi], axis=0)


def _proj_route_body(o_ref, x_ref, wo_ref, ln_ref, gw_ref,
                     a_ref, h_ref, route_ref):
    a = x_ref[...] + jnp.dot(o_ref[...].astype(jnp.bfloat16), wo_ref[...],
                             preferred_element_type=jnp.float32)
    a_ref[...] = a
    ms = jnp.mean(a * a, axis=-1, keepdims=True)
    hn = a * jax.lax.rsqrt(ms + 1e-6) * ln_ref[...]
    h_ref[...] = hn.reshape(2 * BT, D // 2)
    logits = jnp.dot(hn.astype(jnp.bfloat16), gw_ref[...],
                     preferred_element_type=jnp.float32)
    lane = jax.lax.broadcasted_iota(jnp.int32, (BT, 128), 1)
    logits = jnp.where(lane < E, logits, NEG)
    m1 = jnp.max(logits, axis=-1, keepdims=True)
    idx1 = jnp.min(jnp.where(logits == m1, lane, 127), axis=-1, keepdims=True)
    oh1 = (lane == idx1).astype(jnp.float32)
    lm = jnp.where(lane == idx1, NEG, logits)
    m2 = jnp.max(lm, axis=-1, keepdims=True)
    idx2 = jnp.min(jnp.where(lm == m2, lane, 127), axis=-1, keepdims=True)
    oh2 = (lane == idx2).astype(jnp.float32)
    d = jnp.exp(m2 - m1)
    p1 = 1.0 / (1.0 + d)
    p2 = d / (1.0 + d)
    route_ref[...] = (jnp.where(lane == 0, idx1.astype(jnp.float32), 0.0)
                      + jnp.where(lane == 1, idx2.astype(jnp.float32), 0.0)
                      + jnp.where(lane == 2, p1, 0.0)
                      + jnp.where(lane == 3, p2, 0.0))


def _proj_route(o, x2, wo_bf, ln2_w, gate_pad):
    return pl.pallas_call(
        _proj_route_body,
        grid=(NT,),
        in_specs=[
            pl.BlockSpec((BT, D), lambda i: (i, 0)),
            pl.BlockSpec((BT, D), lambda i: (i, 0)),
            pl.BlockSpec((D, D), lambda i: (0, 0)),
            pl.BlockSpec((1, D), lambda i: (0, 0)),
            pl.BlockSpec((D, 128), lambda i: (0, 0)),
        ],
        out_specs=[
            pl.BlockSpec((BT, D), lambda i: (i, 0)),
            pl.BlockSpec((2 * BT, D // 2), lambda i: (i, 0)),
            pl.BlockSpec((BT, 128), lambda i: (i, 0)),
        ],
        out_shape=[
            jax.ShapeDtypeStruct((S, D), jnp.float32),
            jax.ShapeDtypeStruct((2 * S, D // 2), jnp.float32),
            jax.ShapeDtypeStruct((S, 128), jnp.float32),
        ],
    )(o, x2, wo_bf, ln2_w, gate_pad)


def _ranks_body(route_ref, ranks_ref, meta_ref, carry_ref):
    i = pl.program_id(0)

    @pl.when(i == 0)
    def _():
        carry_ref[...] = jnp.zeros((8, 128), jnp.float32)

    lane = jax.lax.broadcasted_iota(jnp.int32, (RT, 128), 1)
    lanef = lane.astype(jnp.float32)
    r = route_ref[...]
    idx0 = r[:, 0:1]
    idx1 = r[:, 1:2]
    oh0 = (lanef == idx0).astype(jnp.float32)
    oh1 = (lanef == idx1).astype(jnp.float32)
    ohb = oh0 + oh1
    row_i = jax.lax.broadcasted_iota(jnp.int32, (RT, RT), 0)
    col_i = jax.lax.broadcasted_iota(jnp.int32, (RT, RT), 1)
    ltri = (col_i <= row_i).astype(jnp.float32)
    incl = jnp.dot(ltri, ohb, preferred_element_type=jnp.float32)
    carry = carry_ref[0:1, :]
    cb = carry + incl - ohb
    rank0 = jnp.sum(cb * oh0, axis=-1, keepdims=True)
    rank1 = jnp.sum(cb * oh1, axis=-1, keepdims=True)
    ranks_ref[...] = (jnp.where(lane == 0, rank0, 0.0)
                      + jnp.where(lane == 1, rank1, 0.0))
    counts = carry + jnp.sum(ohb, axis=0, keepdims=True)
    carry_ref[0:1, :] = counts

    @pl.when(i == NRT - 1)
    def _():
        padded = jnp.floor((counts + (GT - 1)) * (1.0 / GT)) * GT
        ru = jax.lax.broadcasted_iota(jnp.int32, (128, 128), 0)
        cu = jax.lax.broadcasted_iota(jnp.int32, (128, 128), 1)
        sut = (ru < cu).astype(jnp.float32)
        off = jnp.dot(padded, sut, preferred_element_type=jnp.float32)
        ends = off + padded
        sub = jax.lax.broadcasted_iota(jnp.int32, (8, 128), 0)
        meta_ref[...] = (jnp.where(sub == 0, off, 0.0)
                         + jnp.where(sub == 1, ends, 0.0))


def _ranks(route):
    return pl.pallas_call(
        _ranks_body,
        grid=(NRT,),
        in_specs=[pl.BlockSpec((RT, 128), lambda i: (i, 0))],
        out_specs=[
            pl.BlockSpec((RT, 128), lambda i: (i, 0)),
            pl.BlockSpec((8, 128), lambda i: (0, 0)),
        ],
        out_shape=[
            jax.ShapeDtypeStruct((S, 128), jnp.float32),
            jax.ShapeDtypeStruct((8, 128), jnp.float32),
        ],
        scratch_shapes=[pltpu.VMEM((8, 128), jnp.float32)],
    )(route)


def _dest_body(route_ref, ranks_ref, meta_ref, dest_ref):
    lane = jax.lax.broadcasted_iota(jnp.int32, (S, 128), 1)
    lanef = lane.astype(jnp.float32)
    r = route_ref[...]
    idx0 = r[:, 0:1]
    idx1 = r[:, 1:2]
    off_row = meta_ref[0:1, :]
    off0 = jnp.sum((lanef == idx0).astype(jnp.float32) * off_row,
                   axis=-1, keepdims=True)
    off1 = jnp.sum((lanef == idx1).astype(jnp.float32) * off_row,
                   axis=-1, keepdims=True)
    dest0 = off0 + ranks_ref[:, 0:1]
    dest1 = off1 + ranks_ref[:, 1:2]
    dest_ref[...] = (jnp.where(lane == 0, dest0, 0.0)
                     + jnp.where(lane == 1, dest1, 0.0))


def _dest(route, ranks, meta):
    return pl.pallas_call(
        _dest_body,
        grid=(1,),
        in_specs=[
            pl.BlockSpec((S, 128), lambda i: (0, 0)),
            pl.BlockSpec((S, 128), lambda i: (0, 0)),
            pl.BlockSpec((8, 128), lambda i: (0, 0)),
        ],
        out_specs=pl.BlockSpec((S, 128), lambda i: (0, 0)),
        out_shape=jax.ShapeDtypeStruct((S, 128), jnp.float32),
    )(route, ranks, meta)


def _vector_mesh():
    return plsc.VectorSubcoreMesh(core_axis_name="core",
                                  subcore_axis_name="subcore")


HD = D // 2


def _sc_scatter_h(h2, idx2):
    @pl.kernel(out_type=jax.ShapeDtypeStruct((2 * CAP, HD), jnp.float32),
               mesh=_vector_mesh())
    def k(h_hbm, i_hbm, o_hbm):
        def body(h_vmem, i_vmem):
            pltpu.sync_copy(h_vmem, o_hbm.at[i_vmem.at[0]])

        pltpu.emit_pipeline(
            body,
            grid=(K, 2 * S // SCW),
            in_specs=[
                pl.BlockSpec((SCW, HD), index_map=lambda s, c: (c, 0)),
                pl.BlockSpec((1, SCW), index_map=lambda s, c: (s, c)),
            ],
            out_specs=[],
            core_axis_name=("core", "subcore"),
            dimension_semantics=(pltpu.PARALLEL, pltpu.PARALLEL),
        )(h_hbm, i_hbm)

    return k(h2, idx2)


FC = 768


def _group_ffn_body(et_ref, hs_ref, w1_ref, w2_ref, y_ref):
    hb = hs_ref[...].reshape(GT, D).astype(jnp.bfloat16)
    acc = None
    for c in range(F // FC):
        t = jnp.dot(hb, w1_ref[0, :, c * FC:(c + 1) * FC],
                    preferred_element_type=jnp.float32)
        act = t * jax.nn.sigmoid(t)
        part = jnp.dot(act.astype(jnp.bfloat16),
                       w2_ref[0, c * FC:(c + 1) * FC, :],
                       preferred_element_type=jnp.float32)
        acc = part if acc is None else acc + part
    y_ref[...] = acc.reshape(2 * GT, HD)


def _group_ffn(etile, hs, w1_bf, w2_bf):
    grid_spec = pltpu.PrefetchScalarGridSpec(
        num_scalar_prefetch=1,
        grid=(NTILES,),
        in_specs=[
            pl.BlockSpec((2 * GT, HD), lambda i, et: (i, 0)),
            pl.BlockSpec((1, D, F), lambda i, et: (et[i], 0, 0)),
            pl.BlockSpec((1, F, D), lambda i, et: (et[i], 0, 0)),
        ],
        out_specs=pl.BlockSpec((2 * GT, HD), lambda i, et: (i, 0)),
    )
    return pl.pallas_call(
        _group_ffn_body,
        grid_spec=grid_spec,
        out_shape=jax.ShapeDtypeStruct((2 * CAP, HD), jnp.float32),
    )(etile, hs, w1_bf, w2_bf)


def _sc_gather_y(y2, didx):
    n = 2 * S * K

    @pl.kernel(out_type=jax.ShapeDtypeStruct((n, HD), jnp.float32),
               mesh=_vector_mesh())
    def k(y_hbm, i_hbm, o_hbm):
        def body(i_vmem, o_vmem):
            pltpu.sync_copy(y_hbm.at[i_vmem.at[0]], o_vmem)

        pltpu.emit_pipeline(
            body,
            grid=(n // SCW,),
            in_specs=[pl.BlockSpec((1, SCW), index_map=lambda i: (0, i))],
            out_specs=[pl.BlockSpec((SCW, HD), index_map=lambda i: (i, 0))],
            core_axis_name=("core", "subcore"),
            dimension_semantics=(pltpu.PARALLEL,),
        )(i_hbm, o_hbm)

    return k(y2, didx)


def _combine_body(a_ref, yc_ref, route_ref, out_ref):
    lane = jax.lax.broadcasted_iota(jnp.int32, (BT, 128), 1)
    r = route_ref[...]
    p0 = jnp.sum(jnp.where(lane == 2, r, 0.0), axis=-1, keepdims=True)
    p1 = jnp.sum(jnp.where(lane == 3, r, 0.0), axis=-1, keepdims=True)
    yc = yc_ref[...].reshape(BT, 2 * K * HD)
    out_ref[...] = a_ref[...] + p0 * yc[:, :D] + p1 * yc[:, D:]


def _combine(a, yc2, route):
    return pl.pallas_call(
        _combine_body,
        grid=(NT,),
        in_specs=[
            pl.BlockSpec((BT, D), lambda i: (i, 0)),
            pl.BlockSpec((2 * K * BT, HD), lambda i: (i, 0)),
            pl.BlockSpec((BT, 128), lambda i: (i, 0)),
        ],
        out_specs=pl.BlockSpec((BT, D), lambda i: (i, 0)),
        out_shape=jax.ShapeDtypeStruct((S, D), jnp.float32),
    )(a, yc2, route)


def kernel(x, ln1_w, ln2_w, Wq, Wk, Wv, Wo, gate_w, W1, W2):
    x2 = x.reshape(S, D)
    wqkv = jnp.concatenate([Wq, Wk, Wv], axis=1).astype(jnp.bfloat16)
    qkv = _ln_qkv(x2, wqkv, ln1_w.reshape(1, D))
    o = _attn(qkv)
    gate_pad = jnp.pad(gate_w, ((0, 0), (0, 128 - E))).astype(jnp.bfloat16)
    a, h_bf, route = _proj_route(o, x2, Wo.astype(jnp.bfloat16),
                                 ln2_w.reshape(1, D), gate_pad)
    ranks, meta = _ranks(route)
    dest = _dest(route, ranks, meta)
    dest_i = dest[:, :K].astype(jnp.int32)
    ends = meta[1, :E]
    starts = jnp.arange(NTILES, dtype=jnp.float32) * GT
    etile = jnp.minimum(
        jnp.sum((starts[:, None] >= ends[None, :]).astype(jnp.int32), axis=1),
        E - 1).astype(jnp.int32)
    slot = dest_i.T
    idx2 = jnp.stack([2 * slot, 2 * slot + 1], axis=-1).reshape(K, 2 * S)
    hs2 = _sc_scatter_h(h_bf, idx2)
    ys = _group_ffn(etile, hs2, W1.astype(jnp.bfloat16),
                    W2.astype(jnp.bfloat16))
    dflat = dest_i.reshape(S * K)
    didx = jnp.stack([2 * dflat, 2 * dflat + 1], axis=-1).reshape(1, 2 * S * K)
    yc = _sc_gather_y(ys, didx)
    out = _combine(a, yc, route)
    return out.reshape(1, S, D)
